# Initial kernel scaffold; baseline (speedup 1.0000x reference)
import functools

import jax
import jax.numpy as jnp
from jax import lax
from jax.experimental import pallas as pl
from jax.experimental.pallas import tpu as pltpu
from jax.experimental.pallas import tpu_sc as plsc

NC, NS, L = 2, 16, 16
NW = NC * NS

SEQ = 200
BATCH = 16384
EMB = 32

CHUNK = 16
ROWS = CHUNK * SEQ
GATHER_W = 128
IDX_TILES = ROWS // GATHER_W
PER_W = BATCH // NW
N_CHUNKS = PER_W // CHUNK
IDX_ROWS_PER_W = PER_W * SEQ // GATHER_W


def _sc_body(emb_h, idx_h, wb_h, bias_h, out_h,
             idx_v, rows_v, hbuf, wb_v, bias_v, out_v, sem):
    w = lax.axis_index("s") * NC + lax.axis_index("c")

    pltpu.sync_copy(wb_h, wb_v)
    pltpu.sync_copy(bias_h, bias_v)

    lane = lax.iota(jnp.int32, L)
    zero = jnp.zeros((L,), jnp.float32)

    @pl.loop(0, N_CHUNKS)
    def chunk_loop(k):
        row0 = w * IDX_ROWS_PER_W + k * IDX_TILES
        pltpu.sync_copy(idx_h.at[pl.ds(row0, IDX_TILES), :], idx_v)
        copies = []
        for j in range(IDX_TILES):
            copies.append(pltpu.async_copy(
                emb_h.at[idx_v.at[j]],
                rows_v.at[pl.ds(j * GATHER_W, GATHER_W), :],
                sem))
        for cp in copies:
            cp.wait()

        for c in range(CHUNK):
            base = c * SEQ

            def rbody(r, acc, base=base):
                a0, a1 = acc
                a0 = a0 + rows_v[base + r, 0:16]
                a1 = a1 + rows_v[base + r, 16:32]
                return a0, a1

            a0, a1 = lax.fori_loop(0, SEQ, rbody, (zero, zero), unroll=8)
            hbuf[pl.ds(c * EMB, 16)] = a0
            hbuf[pl.ds(c * EMB + 16, 16)] = a1

        yacc = bias_v[...]
        for d in range(EMB):
            col = plsc.load_gather(hbuf, [lane * EMB + d])
            e = jnp.exp(col * (2.0 / SEQ))
            t = 1.0 - 2.0 / (e + 1.0)
            yacc = yacc + t * wb_v[d, 0:16]
        out_v[pl.ds(k * CHUNK, CHUNK)] = yacc

    pltpu.sync_copy(out_v, out_h.at[pl.ds(w * PER_W, PER_W)])


@functools.partial(
    pl.kernel,
    out_type=jax.ShapeDtypeStruct((BATCH,), jnp.float32),
    mesh=plsc.VectorSubcoreMesh(core_axis_name="c", subcore_axis_name="s",
                                num_cores=NC, num_subcores=NS),
    scratch_types=[
        pltpu.VMEM((IDX_TILES, GATHER_W), jnp.int32),
        pltpu.VMEM((ROWS, EMB), jnp.float32),
        pltpu.VMEM((CHUNK * EMB,), jnp.float32),
        pltpu.VMEM((EMB, L), jnp.float32),
        pltpu.VMEM((L,), jnp.float32),
        pltpu.VMEM((PER_W,), jnp.float32),
        pltpu.SemaphoreType.DMA,
    ],
)
def _sc_kernel(emb_h, idx_h, wb_h, bias_h, out_h,
               idx_v, rows_v, hbuf, wb_v, bias_v, out_v, sem):
    _sc_body(emb_h, idx_h, wb_h, bias_h, out_h,
             idx_v, rows_v, hbuf, wb_v, bias_v, out_v, sem)


def kernel(input, emb, W, b):
    idx2d = input.T.reshape(-1, GATHER_W)
    wb = jnp.broadcast_to(W.reshape(EMB, 1), (EMB, L))
    bias = jnp.broadcast_to(b.reshape(1), (L,))
    out = _sc_kernel(emb, idx2d, wb, bias)
    return out.reshape(BATCH, 1)


# speedup vs baseline: 5.3220x; 5.3220x over previous
import functools

import jax
import jax.numpy as jnp
from jax import lax
from jax.experimental import pallas as pl
from jax.experimental.pallas import tpu as pltpu
from jax.experimental.pallas import tpu_sc as plsc

NC, NS, L = 2, 16, 16
NW = NC * NS

SEQ = 200
BATCH = 16384
EMB = 32

CHUNK = 16
ROWS = CHUNK * SEQ
GATHER_W = 128
IDX_TILES = ROWS // GATHER_W
PER_W = BATCH // NW
N_CHUNKS = PER_W // CHUNK


def _sc_body(emb_h, idx_h, wb_h, bias_h, out_h,
             idx_v, rows_v, wb_v, bias_v, out_v, sem):
    w = lax.axis_index("s") * NC + lax.axis_index("c")

    pltpu.sync_copy(wb_h, wb_v)
    pltpu.sync_copy(bias_h, bias_v)

    lane = lax.iota(jnp.int32, L)
    zero = jnp.zeros((L,), jnp.float32)

    @pl.loop(0, N_CHUNKS)
    def chunk_loop(k):
        idx0 = (w * PER_W + k * CHUNK) * SEQ
        pltpu.sync_copy(idx_h.at[pl.ds(idx0, ROWS)], idx_v)
        copies = []
        for j in range(IDX_TILES):
            copies.append(pltpu.async_copy(
                emb_h.at[idx_v.at[pl.ds(j * GATHER_W, GATHER_W)]],
                rows_v.at[pl.ds(j * GATHER_W, GATHER_W), :],
                sem))
        for cp in copies:
            cp.wait()

        w0 = wb_v[0:16]
        w1 = wb_v[16:32]
        yacc = bias_v[...]
        for c in range(CHUNK):
            base = c * SEQ

            def rbody(r, acc, base=base):
                a0, a1 = acc
                a0 = a0 + rows_v[base + r, 0:16]
                a1 = a1 + rows_v[base + r, 16:32]
                return a0, a1

            a0, a1 = lax.fori_loop(0, SEQ, rbody, (zero, zero), unroll=8)
            e0 = jnp.exp(a0 * (2.0 / SEQ))
            e1 = jnp.exp(a1 * (2.0 / SEQ))
            t0 = 1.0 - 2.0 / (e0 + 1.0)
            t1 = 1.0 - 2.0 / (e1 + 1.0)
            total = jnp.sum(t0 * w0 + t1 * w1)
            yacc = yacc + jnp.where(lane == c, total, 0.0)
        out_v[pl.ds(k * CHUNK, CHUNK)] = yacc

    pltpu.sync_copy(out_v, out_h.at[pl.ds(w * PER_W, PER_W)])


@functools.partial(
    pl.kernel,
    out_type=jax.ShapeDtypeStruct((BATCH,), jnp.float32),
    mesh=plsc.VectorSubcoreMesh(core_axis_name="c", subcore_axis_name="s",
                                num_cores=NC, num_subcores=NS),
    compiler_params=pltpu.CompilerParams(needs_layout_passes=False,
                                         use_tc_tiling_on_sc=False),
    scratch_types=[
        pltpu.VMEM((ROWS,), jnp.int32),
        pltpu.VMEM((ROWS, EMB), jnp.float32),
        pltpu.VMEM((EMB,), jnp.float32),
        pltpu.VMEM((L,), jnp.float32),
        pltpu.VMEM((PER_W,), jnp.float32),
        pltpu.SemaphoreType.DMA,
    ],
)
def _sc_kernel(emb_h, idx_h, wb_h, bias_h, out_h,
               idx_v, rows_v, wb_v, bias_v, out_v, sem):
    _sc_body(emb_h, idx_h, wb_h, bias_h, out_h,
             idx_v, rows_v, wb_v, bias_v, out_v, sem)


def kernel(input, emb, W, b):
    idx1d = input.T.reshape(-1)
    wb = W.reshape(EMB)
    bias = jnp.broadcast_to(b.reshape(1), (L,))
    out = _sc_kernel(emb, idx1d, wb, bias)
    return out.reshape(BATCH, 1)
